# baseline (speedup 1.0000x reference)
import functools

import jax
import jax.numpy as jnp
from jax import lax
from jax.experimental import pallas as pl
from jax.experimental.pallas import tpu as pltpu
from jax.experimental.pallas import tpu_sc as plsc

B, N, C = 8, 32768, 64
M = N // 4
NC, NS, L = 2, 16, 16
NW = NC * NS
RPW = B * M // NW
CHUNK = 128
NCHUNK = RPW // CHUNK
WPB = M // RPW


def _body(pos_hbm, feat_hbm, fps_hbm, pos_out, feat_out,
          raw_v, idx_v, feat_buf, pos_tile, pos_buf, ssem, gsem, osem):
    wid = lax.axis_index("s") * NC + lax.axis_index("c")
    base = wid * RPW
    batch = wid // WPB
    off = batch * N

    stage = pltpu.async_copy(
        pos_hbm.at[pl.ds(batch * (N * 3), N * 3)], pos_tile, ssem)

    pltpu.sync_copy(fps_hbm.at[wid], raw_v)
    for t in range(RPW // L):
        sl = pl.ds(t * L, L)
        idx_v[sl] = raw_v[sl] + off

    waits = [None, None]
    pltpu.async_copy(feat_hbm.at[idx_v.at[pl.ds(0, CHUNK)]], feat_buf.at[0],
                     gsem)
    for i in range(NCHUNK):
        b = i % 2
        pltpu.make_async_copy(feat_hbm.at[idx_v.at[pl.ds(i * CHUNK, CHUNK)]],
                              feat_buf.at[b], gsem).wait()
        if i + 1 < NCHUNK:
            nb = (i + 1) % 2
            if waits[nb] is not None:
                waits[nb].wait()
                waits[nb] = None
            pltpu.async_copy(
                feat_hbm.at[idx_v.at[pl.ds((i + 1) * CHUNK, CHUNK)]],
                feat_buf.at[nb], gsem)
        waits[b] = pltpu.async_copy(
            feat_buf.at[b], feat_out.at[pl.ds(base + i * CHUNK, CHUNK)], osem)

    stage.wait()
    lane3 = lax.iota(jnp.int32, L) * 3
    for t in range(RPW // L):
        rid3 = raw_v[pl.ds(t * L, L)] * 3
        dst = lane3 + (t * (L * 3))
        for k in range(3):
            vals = plsc.load_gather(pos_tile, [rid3 + k])
            plsc.store_scatter(pos_buf, [dst + k], vals)
    pltpu.sync_copy(pos_buf, pos_out.at[pl.ds(base * 3, RPW * 3)])

    for w in waits:
        if w is not None:
            w.wait()


@jax.jit
def _gather(pos_flat, feat_flat, fps_r):
    mesh = plsc.VectorSubcoreMesh(core_axis_name="c", subcore_axis_name="s")
    f = functools.partial(
        pl.kernel, mesh=mesh,
        out_type=(jax.ShapeDtypeStruct((B * M * 3,), jnp.float32),
                  jax.ShapeDtypeStruct((B * M, C), jnp.float32)),
        scratch_types=[
            pltpu.VMEM((RPW,), jnp.int32),
            pltpu.VMEM((RPW,), jnp.int32),
            pltpu.VMEM((2, CHUNK, C), jnp.float32),
            pltpu.VMEM((N * 3,), jnp.float32),
            pltpu.VMEM((RPW * 3,), jnp.float32),
            pltpu.SemaphoreType.DMA,
            pltpu.SemaphoreType.DMA,
            pltpu.SemaphoreType.DMA,
        ],
        compiler_params=pltpu.CompilerParams(use_tc_tiling_on_sc=False,
                                             needs_layout_passes=False),
    )(_body)
    return f(pos_flat, feat_flat, fps_r)


def kernel(pos, feat, fps_preprocess):
    pos_flat = pos.reshape(B * N * 3)
    feat_flat = feat.reshape(B * N, C)
    fps_r = fps_preprocess.reshape(NW, RPW)
    pos_ds, feat_ds = _gather(pos_flat, feat_flat, fps_r)
    return pos_ds.reshape(B, M, 3), feat_ds.reshape(B, M, C)


# speedup vs baseline: 4.0473x; 4.0473x over previous
import functools

import jax
import jax.numpy as jnp
from jax import lax
from jax.experimental import pallas as pl
from jax.experimental.pallas import tpu as pltpu
from jax.experimental.pallas import tpu_sc as plsc

B, N, C = 8, 32768, 64
M = N // 4
NC, NS, L = 2, 16, 16
NW = NC * NS
CPW = C // (NW // B)
NPOS = 3 * B


def _body(feat_t, pos_t, fps, out_t, pos_out,
          idx_v, row0, row1, ob0, ob1, s0, s1, o0, o1):
    w = lax.axis_index("s") * NC + lax.axis_index("c")
    b = w // (NW // B)
    cbase = (w % (NW // B)) * CPW
    rows = [row0, row1]
    obs = [ob0, ob1]
    ssems = [s0, s1]
    osems = [o0, o1]

    pltpu.sync_copy(fps.at[b], idx_v)
    pltpu.async_copy(feat_t.at[b, cbase], row0, s0)
    owaits = [None, None]
    for t in range(CPW):
        u = t % 2
        pltpu.make_async_copy(feat_t.at[b, cbase + t], rows[u],
                              ssems[u]).wait()
        if t + 1 < CPW:
            nu = (t + 1) % 2
            pltpu.async_copy(feat_t.at[b, cbase + t + 1], rows[nu],
                             ssems[nu])
        if owaits[u] is not None:
            owaits[u].wait()
            owaits[u] = None

        @pl.loop(0, M // L, unroll=8)
        def _g(g, u=u):
            sl = pl.ds(g * L, L)
            obs[u][sl] = plsc.load_gather(rows[u], [idx_v[sl]])

        owaits[u] = pltpu.async_copy(obs[u], out_t.at[b, cbase + t],
                                     osems[u])
    for wv in owaits:
        if wv is not None:
            wv.wait()

    @pl.when(w < NPOS)
    def _pos():
        k = w // B
        b2 = w % B
        pltpu.sync_copy(fps.at[b2], idx_v)
        pltpu.sync_copy(pos_t.at[k, b2], row0)

        @pl.loop(0, M // L, unroll=8)
        def _g2(g):
            sl = pl.ds(g * L, L)
            ob0[sl] = plsc.load_gather(row0, [idx_v[sl]])

        pltpu.sync_copy(ob0, pos_out.at[k, b2])


@jax.jit
def _sc_gather(feat_t, pos_t, fps):
    mesh = plsc.VectorSubcoreMesh(core_axis_name="c", subcore_axis_name="s")
    f = functools.partial(
        pl.kernel, mesh=mesh,
        out_type=(jax.ShapeDtypeStruct((B, C, M), jnp.float32),
                  jax.ShapeDtypeStruct((3, B, M), jnp.float32)),
        scratch_types=[
            pltpu.VMEM((M,), jnp.int32),
            pltpu.VMEM((N,), jnp.float32),
            pltpu.VMEM((N,), jnp.float32),
            pltpu.VMEM((M,), jnp.float32),
            pltpu.VMEM((M,), jnp.float32),
            pltpu.SemaphoreType.DMA,
            pltpu.SemaphoreType.DMA,
            pltpu.SemaphoreType.DMA,
            pltpu.SemaphoreType.DMA,
        ],
        compiler_params=pltpu.CompilerParams(use_tc_tiling_on_sc=True,
                                             needs_layout_passes=False),
    )(_body)
    return f(feat_t, pos_t, fps)


def kernel(pos, feat, fps_preprocess):
    feat_t = jnp.transpose(feat, (0, 2, 1))
    pos_t = jnp.transpose(pos, (2, 0, 1))
    out_t, pos_out_t = _sc_gather(feat_t, pos_t, fps_preprocess)
    pos_ds = jnp.transpose(pos_out_t, (1, 2, 0))
    feat_ds = jnp.transpose(out_t, (0, 2, 1))
    return pos_ds, feat_ds


# speedup vs baseline: 6.6312x; 1.6384x over previous
import functools

import jax
import jax.numpy as jnp
from jax import lax
from jax.experimental import pallas as pl
from jax.experimental.pallas import tpu as pltpu
from jax.experimental.pallas import tpu_sc as plsc

B, N, C = 8, 32768, 64
M = N // 4
NC, NS, L = 2, 16, 16
NW = NC * NS
CPW = C // (NW // B)
NPOS = 3 * B


def _body(feat_t, pos_t, fps, out_t, pos_out,
          idx_v, row0, row1, ob0, ob1, s0, s1, o0, o1):
    w = lax.axis_index("s") * NC + lax.axis_index("c")
    b = w // (NW // B)
    cbase = (w % (NW // B)) * CPW
    rows = [row0, row1]
    obs = [ob0, ob1]
    ssems = [s0, s1]
    osems = [o0, o1]

    pltpu.sync_copy(fps.at[b], idx_v)
    pltpu.async_copy(feat_t.at[b, cbase], row0, s0)
    owaits = [None, None]
    for t in range(CPW):
        u = t % 2
        pltpu.make_async_copy(feat_t.at[b, cbase + t], rows[u],
                              ssems[u]).wait()
        if t + 1 < CPW:
            nu = (t + 1) % 2
            pltpu.async_copy(feat_t.at[b, cbase + t + 1], rows[nu],
                             ssems[nu])
        if owaits[u] is not None:
            owaits[u].wait()
            owaits[u] = None

        @plsc.parallel_loop(0, M // L, unroll=8)
        def _g(g, u=u):
            sl = pl.ds(g * L, L)
            obs[u][sl] = plsc.load_gather(rows[u], [idx_v[sl]])

        owaits[u] = pltpu.async_copy(obs[u], out_t.at[b, cbase + t],
                                     osems[u])
    for wv in owaits:
        if wv is not None:
            wv.wait()

    @pl.when(w < NPOS)
    def _pos():
        k = w // B
        b2 = w % B
        pltpu.sync_copy(fps.at[b2], idx_v)
        pltpu.sync_copy(pos_t.at[k, b2], row0)

        @plsc.parallel_loop(0, M // L, unroll=8)
        def _g2(g):
            sl = pl.ds(g * L, L)
            ob0[sl] = plsc.load_gather(row0, [idx_v[sl]])

        pltpu.sync_copy(ob0, pos_out.at[k, b2])


@jax.jit
def _sc_gather(feat_t, pos_t, fps):
    mesh = plsc.VectorSubcoreMesh(core_axis_name="c", subcore_axis_name="s")
    f = functools.partial(
        pl.kernel, mesh=mesh,
        out_type=(jax.ShapeDtypeStruct((B, C, M), jnp.float32),
                  jax.ShapeDtypeStruct((3, B, M), jnp.float32)),
        scratch_types=[
            pltpu.VMEM((M,), jnp.int32),
            pltpu.VMEM((N,), jnp.float32),
            pltpu.VMEM((N,), jnp.float32),
            pltpu.VMEM((M,), jnp.float32),
            pltpu.VMEM((M,), jnp.float32),
            pltpu.SemaphoreType.DMA,
            pltpu.SemaphoreType.DMA,
            pltpu.SemaphoreType.DMA,
            pltpu.SemaphoreType.DMA,
        ],
        compiler_params=pltpu.CompilerParams(use_tc_tiling_on_sc=True,
                                             needs_layout_passes=False),
    )(_body)
    return f(feat_t, pos_t, fps)


def kernel(pos, feat, fps_preprocess):
    feat_t = jnp.transpose(feat, (0, 2, 1))
    pos_t = jnp.transpose(pos, (2, 0, 1))
    out_t, pos_out_t = _sc_gather(feat_t, pos_t, fps_preprocess)
    pos_ds = jnp.transpose(pos_out_t, (1, 2, 0))
    feat_ds = jnp.transpose(out_t, (0, 2, 1))
    return pos_ds, feat_ds


# speedup vs baseline: 6.8964x; 1.0400x over previous
import functools

import jax
import jax.numpy as jnp
from jax import lax
from jax.experimental import pallas as pl
from jax.experimental.pallas import tpu as pltpu
from jax.experimental.pallas import tpu_sc as plsc

B, N, C = 8, 32768, 64
M = N // 4
NC, NS, L = 2, 16, 16
NW = NC * NS
CPW = C // (NW // B)
NPOS = 3 * B


def _body(feat_t, pos_t, fps, out_t, pos_out,
          idx_v, row0, row1, prow, ob0, ob1, s0, s1, o0, o1, ps):
    w = lax.axis_index("s") * NC + lax.axis_index("c")
    b = w // (NW // B)
    cbase = (w % (NW // B)) * CPW
    k = w % (NW // B)
    rows = [row0, row1]
    obs = [ob0, ob1]
    ssems = [s0, s1]
    osems = [o0, o1]

    pltpu.sync_copy(fps.at[b], idx_v)
    pltpu.async_copy(feat_t.at[b, cbase], row0, s0)

    @pl.when(k < 3)
    def _pos_prefetch():
        pltpu.async_copy(pos_t.at[k, b], prow, ps)

    owaits = [None, None]
    for t in range(CPW):
        u = t % 2
        pltpu.make_async_copy(feat_t.at[b, cbase + t], rows[u],
                              ssems[u]).wait()
        if t + 1 < CPW:
            nu = (t + 1) % 2
            pltpu.async_copy(feat_t.at[b, cbase + t + 1], rows[nu],
                             ssems[nu])
        if owaits[u] is not None:
            owaits[u].wait()
            owaits[u] = None

        @plsc.parallel_loop(0, M // L, unroll=8)
        def _g(g, u=u):
            sl = pl.ds(g * L, L)
            obs[u][sl] = plsc.load_gather(rows[u], [idx_v[sl]])

        owaits[u] = pltpu.async_copy(obs[u], out_t.at[b, cbase + t],
                                     osems[u])
    for wv in owaits:
        if wv is not None:
            wv.wait()

    @pl.when(k < 3)
    def _pos():
        pltpu.make_async_copy(pos_t.at[k, b], prow, ps).wait()

        @plsc.parallel_loop(0, M // L, unroll=8)
        def _g2(g):
            sl = pl.ds(g * L, L)
            ob0[sl] = plsc.load_gather(prow, [idx_v[sl]])

        pltpu.sync_copy(ob0, pos_out.at[k, b])


@jax.jit
def _sc_gather(feat_t, pos_t, fps):
    mesh = plsc.VectorSubcoreMesh(core_axis_name="c", subcore_axis_name="s")
    f = functools.partial(
        pl.kernel, mesh=mesh,
        out_type=(jax.ShapeDtypeStruct((B, C, M), jnp.float32),
                  jax.ShapeDtypeStruct((3, B, M), jnp.float32)),
        scratch_types=[
            pltpu.VMEM((M,), jnp.int32),
            pltpu.VMEM((N,), jnp.float32),
            pltpu.VMEM((N,), jnp.float32),
            pltpu.VMEM((N,), jnp.float32),
            pltpu.VMEM((M,), jnp.float32),
            pltpu.VMEM((M,), jnp.float32),
            pltpu.SemaphoreType.DMA,
            pltpu.SemaphoreType.DMA,
            pltpu.SemaphoreType.DMA,
            pltpu.SemaphoreType.DMA,
            pltpu.SemaphoreType.DMA,
        ],
        compiler_params=pltpu.CompilerParams(use_tc_tiling_on_sc=True,
                                             needs_layout_passes=False),
    )(_body)
    return f(feat_t, pos_t, fps)


def kernel(pos, feat, fps_preprocess):
    feat_t = jnp.transpose(feat, (0, 2, 1))
    pos_t = jnp.transpose(pos, (2, 0, 1))
    out_t, pos_out_t = _sc_gather(feat_t, pos_t, fps_preprocess)
    pos_ds = jnp.transpose(pos_out_t, (1, 2, 0))
    feat_ds = jnp.transpose(out_t, (0, 2, 1))
    return pos_ds, feat_ds
